# baseline (speedup 1.0000x reference)
import jax
import jax.numpy as jnp
from jax.experimental import pallas as pl
from jax.experimental.pallas import tpu as pltpu

N = 512
BI = 128
F32 = jnp.float32


def _edge_block(piT_blk, pjT, e_blk, et_blk, uv, w3, bai, bao, a_blk):
    ee = jnp.stack([e_blk, et_blk])
    edge = jax.lax.dot_general(uv, ee, (((1,), (0,)), ((), ())),
                               preferred_element_type=F32,
                               precision=jax.lax.Precision.HIGHEST)
    T = jax.nn.relu(piT_blk[:, :, None] + pjT[:, None, :] + edge)
    Z = jax.lax.dot_general(w3, T, (((1,), (0,)), ((), ())),
                            preferred_element_type=F32,
                            precision=jax.lax.Precision.HIGHEST)
    mask = (a_blk != 0.0).astype(F32)
    wi = mask * jax.nn.sigmoid(Z[0] + bai)
    wo = mask * jax.nn.sigmoid(Z[1] + bao)
    return T, Z, wi, wo


def _layer1_kernel(x_ref, e_row_ref, e_col_ref, a_ref,
                   wsi_ref, wsj_ref, uv_ref, bs_ref,
                   w3_ref, bai_ref, bao_ref, be_ref,
                   min_ref, mout_ref, e1_ref, e1t_ref):
    i = pl.program_id(0)
    x = x_ref[...]
    xb = x_ref[pl.ds(i * BI, BI), :]
    piT = jax.lax.dot_general(wsi_ref[...], xb, (((0,), (1,)), ((), ())),
                              preferred_element_type=F32, precision=jax.lax.Precision.HIGHEST) + bs_ref[...]
    pjT = jax.lax.dot_general(wsj_ref[...], x, (((0,), (1,)), ((), ())),
                              preferred_element_type=F32, precision=jax.lax.Precision.HIGHEST)

    e_blk = e_row_ref[...]
    et_blk = e_col_ref[...].T
    T, Z, wi, wo = _edge_block(piT, pjT, e_blk, et_blk,
                               uv_ref[...], w3_ref[...],
                               bai_ref[...], bao_ref[...], a_ref[...])

    min_ref[...] = jnp.sum(T * wi[None, :, :], axis=2)
    mo = jnp.sum(T * wo[None, :, :], axis=1)

    @pl.when(i == 0)
    def _():
        mout_ref[...] = mo

    @pl.when(i > 0)
    def _():
        mout_ref[...] = mout_ref[...] + mo

    e_new = Z[2] + be_ref[...]
    e1_ref[...] = e_new
    e1t_ref[...] = e_new.T


def _layer2_kernel(x_ref, min1_ref, mout1_ref, wn1_ref, bn1_ref,
                   e1_ref, e1t_ref, a_ref,
                   wsi_ref, wsj_ref, uv_ref, bs_ref,
                   w3_ref, bai_ref, bao_ref,
                   wn2_ref, bn2_ref, wd_ref, bd_ref,
                   out_ref,
                   h1_scr, piT_scr, pjT_scr, min2_scr, mout2_scr):
    i = pl.program_id(0)
    nsteps = pl.num_programs(0)

    @pl.when(i == 0)
    def _():
        wn1 = wn1_ref[...]
        h1 = (
            jnp.dot(x_ref[...], wn1[:64], preferred_element_type=F32, precision=jax.lax.Precision.HIGHEST)
            + jax.lax.dot_general(min1_ref[...], wn1[64:96],
                                  (((0,), (0,)), ((), ())),
                                  preferred_element_type=F32, precision=jax.lax.Precision.HIGHEST)
            + jax.lax.dot_general(mout1_ref[...], wn1[96:128],
                                  (((0,), (0,)), ((), ())),
                                  preferred_element_type=F32, precision=jax.lax.Precision.HIGHEST)
            + bn1_ref[...]
        )
        h1_scr[...] = h1
        piT_scr[...] = jax.lax.dot_general(wsi_ref[...], h1,
                                           (((0,), (1,)), ((), ())),
                                           preferred_element_type=F32, precision=jax.lax.Precision.HIGHEST) + bs_ref[...]
        pjT_scr[...] = jax.lax.dot_general(wsj_ref[...], h1,
                                           (((0,), (1,)), ((), ())),
                                           preferred_element_type=F32, precision=jax.lax.Precision.HIGHEST)

    piT_blk = piT_scr[:, pl.ds(i * BI, BI)]
    T, _, wi, wo = _edge_block(piT_blk, pjT_scr[...],
                               e1_ref[...], e1t_ref[...],
                               uv_ref[...], w3_ref[...],
                               bai_ref[...], bao_ref[...], a_ref[...])

    min2_scr[:, pl.ds(i * BI, BI)] = jnp.sum(T * wi[None, :, :], axis=2)
    mo = jnp.sum(T * wo[None, :, :], axis=1)

    @pl.when(i == 0)
    def _():
        mout2_scr[...] = mo

    @pl.when(i > 0)
    def _():
        mout2_scr[...] = mout2_scr[...] + mo

    @pl.when(i == nsteps - 1)
    def _():
        wn2 = wn2_ref[...]
        x2 = (
            jnp.dot(h1_scr[...], wn2[:240], preferred_element_type=F32, precision=jax.lax.Precision.HIGHEST)
            + jax.lax.dot_general(min2_scr[...], wn2[240:272],
                                  (((0,), (0,)), ((), ())),
                                  preferred_element_type=F32, precision=jax.lax.Precision.HIGHEST)
            + jax.lax.dot_general(mout2_scr[...], wn2[272:304],
                                  (((0,), (0,)), ((), ())),
                                  preferred_element_type=F32, precision=jax.lax.Precision.HIGHEST)
            + bn2_ref[...]
        )
        out_ref[...] = jnp.dot(x2, wd_ref[...],
                               preferred_element_type=F32, precision=jax.lax.Precision.HIGHEST) + bd_ref[...]


def _full(shape):
    return pl.BlockSpec(shape, lambda i: tuple(0 for _ in shape))


def kernel(x, a, e, Ws1, bs1, Wai1, bai1, Wao1, bao1, Wn1, bn1, We1, be1,
           Ws2, bs2, Wai2, bai2, Wao2, bao2, Wn2, bn2, We2, be2, Wd, bd):
    f = x.shape[-1]
    x2d = x.reshape(N, f)
    a2d = a.reshape(N, N)
    e2d = e.reshape(N, N)
    grid = (N // BI,)

    def prep(Ws, bs, Wai, bai, Wao, bao, d, We=None):
        uv = jnp.stack([Ws[2 * d], Ws[2 * d + 1]], axis=1)
        rows = [Wai[:, 0], Wao[:, 0]] + ([We[:, 0]] if We is not None else [])
        w3 = jnp.stack(rows, axis=0)
        return (Ws[:d], Ws[d:2 * d], uv, bs.reshape(32, 1),
                w3, bai.reshape(1, 1), bao.reshape(1, 1))

    w1 = prep(Ws1, bs1, Wai1, bai1, Wao1, bao1, f, We1)
    w2 = prep(Ws2, bs2, Wai2, bai2, Wao2, bao2, 240)

    row_spec = pl.BlockSpec((BI, N), lambda i: (i, 0))
    col_spec = pl.BlockSpec((N, BI), lambda i: (0, i))
    cblk_spec = pl.BlockSpec((32, BI), lambda i: (0, i))

    min1, mout1, e1, e1t = pl.pallas_call(
        _layer1_kernel,
        grid=grid,
        in_specs=[
            _full((N, f)), row_spec, col_spec, row_spec,
            _full((f, 32)), _full((f, 32)),
            _full((32, 2)), _full((32, 1)),
            _full((3, 32)), _full((1, 1)), _full((1, 1)), _full((1, 1)),
        ],
        out_specs=[cblk_spec, _full((32, N)), row_spec, col_spec],
        out_shape=[
            jax.ShapeDtypeStruct((32, N), F32),
            jax.ShapeDtypeStruct((32, N), F32),
            jax.ShapeDtypeStruct((N, N), F32),
            jax.ShapeDtypeStruct((N, N), F32),
        ],
    )(x2d, e2d, e2d, a2d,
      w1[0], w1[1], w1[2], w1[3], w1[4], w1[5], w1[6],
      be1.reshape(1, 1))

    out = pl.pallas_call(
        _layer2_kernel,
        grid=grid,
        in_specs=[
            _full((N, f)), _full((32, N)), _full((32, N)),
            _full((f + 64, 240)), _full((1, 240)),
            row_spec, row_spec, row_spec,
            _full((240, 32)), _full((240, 32)),
            _full((32, 2)), _full((32, 1)),
            _full((2, 32)), _full((1, 1)), _full((1, 1)),
            _full((304, 240)), _full((1, 240)),
            _full((240, 240)), _full((1, 240)),
        ],
        out_specs=_full((N, 240)),
        out_shape=jax.ShapeDtypeStruct((N, 240), F32),
        scratch_shapes=[
            pltpu.VMEM((N, 240), F32),
            pltpu.VMEM((32, N), F32),
            pltpu.VMEM((32, N), F32),
            pltpu.VMEM((32, N), F32),
            pltpu.VMEM((32, N), F32),
        ],
    )(x2d, min1, mout1, Wn1, bn1.reshape(1, 240),
      e1, e1t, a2d,
      w2[0], w2[1], w2[2], w2[3], w2[4], w2[5], w2[6],
      Wn2, bn2.reshape(1, 240), Wd, bd.reshape(1, 240))

    return out.reshape(1, N, 240)


# speedup vs baseline: 1.6749x; 1.6749x over previous
import jax
import jax.numpy as jnp
from jax.experimental import pallas as pl
from jax.experimental.pallas import tpu as pltpu

N = 512
BI = 128
F32 = jnp.float32


def _edge_block(piT_blk, pjT, e_blk, et_blk, uv, w3, bai, bao, a_blk):
    T = jax.nn.relu(
        piT_blk[:, :, None]
        + pjT[:, None, :]
        + e_blk[None, :, :] * uv[:, 0:1, None]
        + et_blk[None, :, :] * uv[:, 1:2, None]
    )
    zi = jnp.sum(T * w3[:, 0:1, None], axis=0) + bai
    zo = jnp.sum(T * w3[:, 1:2, None], axis=0) + bao
    en = (jnp.sum(T * w3[:, 2:3, None], axis=0) if w3.shape[1] == 3 else None)
    mask = (a_blk != 0.0).astype(F32)
    wi = mask * jax.nn.sigmoid(zi)
    wo = mask * jax.nn.sigmoid(zo)
    return T, en, wi, wo


def _layer1_kernel(x_ref, e_row_ref, e_col_ref, a_ref,
                   wsi_ref, wsj_ref, uv_ref, bs_ref,
                   w3_ref, bai_ref, bao_ref, be_ref,
                   min_ref, mout_ref, e1_ref, e1t_ref):
    i = pl.program_id(0)
    x = x_ref[...]
    xb = x_ref[pl.ds(i * BI, BI), :]
    piT = jax.lax.dot_general(wsi_ref[...], xb, (((0,), (1,)), ((), ())),
                              preferred_element_type=F32, precision=jax.lax.Precision.HIGHEST) + bs_ref[...]
    pjT = jax.lax.dot_general(wsj_ref[...], x, (((0,), (1,)), ((), ())),
                              preferred_element_type=F32, precision=jax.lax.Precision.HIGHEST)

    e_blk = e_row_ref[...]
    et_blk = e_col_ref[...].T
    T, en, wi, wo = _edge_block(piT, pjT, e_blk, et_blk,
                                uv_ref[...], w3_ref[...],
                                bai_ref[...], bao_ref[...], a_ref[...])

    min_ref[...] = jnp.sum(T * wi[None, :, :], axis=2)
    mo = jnp.sum(T * wo[None, :, :], axis=1)

    @pl.when(i == 0)
    def _():
        mout_ref[...] = mo

    @pl.when(i > 0)
    def _():
        mout_ref[...] = mout_ref[...] + mo

    e_new = en + be_ref[...]
    e1_ref[...] = e_new
    e1t_ref[...] = e_new.T


def _layer2_kernel(x_ref, min1_ref, mout1_ref, wn1_ref, bn1_ref,
                   e1_ref, e1t_ref, a_ref,
                   wsi_ref, wsj_ref, uv_ref, bs_ref,
                   w3_ref, bai_ref, bao_ref,
                   wn2_ref, bn2_ref, wd_ref, bd_ref,
                   out_ref,
                   h1_scr, piT_scr, pjT_scr, min2_scr, mout2_scr):
    i = pl.program_id(0)
    nsteps = pl.num_programs(0)

    @pl.when(i == 0)
    def _():
        wn1 = wn1_ref[...]
        h1 = (
            jnp.dot(x_ref[...], wn1[:64], preferred_element_type=F32, precision=jax.lax.Precision.HIGHEST)
            + jax.lax.dot_general(min1_ref[...], wn1[64:96],
                                  (((0,), (0,)), ((), ())),
                                  preferred_element_type=F32, precision=jax.lax.Precision.HIGHEST)
            + jax.lax.dot_general(mout1_ref[...], wn1[96:128],
                                  (((0,), (0,)), ((), ())),
                                  preferred_element_type=F32, precision=jax.lax.Precision.HIGHEST)
            + bn1_ref[...]
        )
        h1_scr[...] = h1
        piT_scr[...] = jax.lax.dot_general(wsi_ref[...], h1,
                                           (((0,), (1,)), ((), ())),
                                           preferred_element_type=F32, precision=jax.lax.Precision.HIGHEST) + bs_ref[...]
        pjT_scr[...] = jax.lax.dot_general(wsj_ref[...], h1,
                                           (((0,), (1,)), ((), ())),
                                           preferred_element_type=F32, precision=jax.lax.Precision.HIGHEST)

    piT_blk = piT_scr[:, pl.ds(i * BI, BI)]
    T, _, wi, wo = _edge_block(piT_blk, pjT_scr[...],
                               e1_ref[...], e1t_ref[...],
                               uv_ref[...], w3_ref[...],
                               bai_ref[...], bao_ref[...], a_ref[...])

    min2_scr[:, pl.ds(i * BI, BI)] = jnp.sum(T * wi[None, :, :], axis=2)
    mo = jnp.sum(T * wo[None, :, :], axis=1)

    @pl.when(i == 0)
    def _():
        mout2_scr[...] = mo

    @pl.when(i > 0)
    def _():
        mout2_scr[...] = mout2_scr[...] + mo

    @pl.when(i == nsteps - 1)
    def _():
        wn2 = wn2_ref[...]
        x2 = (
            jnp.dot(h1_scr[...], wn2[:240], preferred_element_type=F32, precision=jax.lax.Precision.HIGHEST)
            + jax.lax.dot_general(min2_scr[...], wn2[240:272],
                                  (((0,), (0,)), ((), ())),
                                  preferred_element_type=F32, precision=jax.lax.Precision.HIGHEST)
            + jax.lax.dot_general(mout2_scr[...], wn2[272:304],
                                  (((0,), (0,)), ((), ())),
                                  preferred_element_type=F32, precision=jax.lax.Precision.HIGHEST)
            + bn2_ref[...]
        )
        out_ref[...] = jnp.dot(x2, wd_ref[...],
                               preferred_element_type=F32, precision=jax.lax.Precision.HIGHEST) + bd_ref[...]


def _full(shape):
    return pl.BlockSpec(shape, lambda i: tuple(0 for _ in shape))


def kernel(x, a, e, Ws1, bs1, Wai1, bai1, Wao1, bao1, Wn1, bn1, We1, be1,
           Ws2, bs2, Wai2, bai2, Wao2, bao2, Wn2, bn2, We2, be2, Wd, bd):
    f = x.shape[-1]
    x2d = x.reshape(N, f)
    a2d = a.reshape(N, N)
    e2d = e.reshape(N, N)
    grid = (N // BI,)

    def prep(Ws, bs, Wai, bai, Wao, bao, d, We=None):
        uv = jnp.stack([Ws[2 * d], Ws[2 * d + 1]], axis=1)
        rows = [Wai[:, 0], Wao[:, 0]] + ([We[:, 0]] if We is not None else [])
        w3 = jnp.stack(rows, axis=1)
        return (Ws[:d], Ws[d:2 * d], uv, bs.reshape(32, 1),
                w3, bai.reshape(1, 1), bao.reshape(1, 1))

    w1 = prep(Ws1, bs1, Wai1, bai1, Wao1, bao1, f, We1)
    w2 = prep(Ws2, bs2, Wai2, bai2, Wao2, bao2, 240)

    row_spec = pl.BlockSpec((BI, N), lambda i: (i, 0))
    col_spec = pl.BlockSpec((N, BI), lambda i: (0, i))
    cblk_spec = pl.BlockSpec((32, BI), lambda i: (0, i))

    min1, mout1, e1, e1t = pl.pallas_call(
        _layer1_kernel,
        grid=grid,
        in_specs=[
            _full((N, f)), row_spec, col_spec, row_spec,
            _full((f, 32)), _full((f, 32)),
            _full((32, 2)), _full((32, 1)),
            _full((32, 3)), _full((1, 1)), _full((1, 1)), _full((1, 1)),
        ],
        out_specs=[cblk_spec, _full((32, N)), row_spec, col_spec],
        out_shape=[
            jax.ShapeDtypeStruct((32, N), F32),
            jax.ShapeDtypeStruct((32, N), F32),
            jax.ShapeDtypeStruct((N, N), F32),
            jax.ShapeDtypeStruct((N, N), F32),
        ],
    )(x2d, e2d, e2d, a2d,
      w1[0], w1[1], w1[2], w1[3], w1[4], w1[5], w1[6],
      be1.reshape(1, 1))

    out = pl.pallas_call(
        _layer2_kernel,
        grid=grid,
        in_specs=[
            _full((N, f)), _full((32, N)), _full((32, N)),
            _full((f + 64, 240)), _full((1, 240)),
            row_spec, row_spec, row_spec,
            _full((240, 32)), _full((240, 32)),
            _full((32, 2)), _full((32, 1)),
            _full((32, 2)), _full((1, 1)), _full((1, 1)),
            _full((304, 240)), _full((1, 240)),
            _full((240, 240)), _full((1, 240)),
        ],
        out_specs=_full((N, 240)),
        out_shape=jax.ShapeDtypeStruct((N, 240), F32),
        scratch_shapes=[
            pltpu.VMEM((N, 240), F32),
            pltpu.VMEM((32, N), F32),
            pltpu.VMEM((32, N), F32),
            pltpu.VMEM((32, N), F32),
            pltpu.VMEM((32, N), F32),
        ],
    )(x2d, min1, mout1, Wn1, bn1.reshape(1, 240),
      e1, e1t, a2d,
      w2[0], w2[1], w2[2], w2[3], w2[4], w2[5], w2[6],
      Wn2, bn2.reshape(1, 240), Wd, bd.reshape(1, 240))

    return out.reshape(1, N, 240)


# speedup vs baseline: 1.8194x; 1.0863x over previous
import jax
import jax.numpy as jnp
from jax.experimental import pallas as pl
from jax.experimental.pallas import tpu as pltpu

N = 512
BI = 128
F32 = jnp.float32


def _edge_block(piT_blk, pjT, e_blk, et_blk, u, v, wai, bai, wao, bao, a_blk):
    T = jax.nn.relu(
        piT_blk[:, :, None]
        + pjT[:, None, :]
        + e_blk[None, :, :] * u
        + et_blk[None, :, :] * v
    )
    zi = jnp.sum(T * wai, axis=0) + bai
    zo = jnp.sum(T * wao, axis=0) + bao
    mask = (a_blk != 0.0).astype(F32)
    wi = mask * jax.nn.sigmoid(zi)
    wo = mask * jax.nn.sigmoid(zo)
    return T, wi, wo


def _layer1_kernel(x_ref, e_row_ref, e_col_ref, a_ref,
                   wsi_ref, wsj_ref, u_ref, v_ref, bs_ref,
                   wai_ref, bai_ref, wao_ref, bao_ref, we_ref, be_ref,
                   min_ref, mout_ref, e1_ref, e1t_ref):
    i = pl.program_id(0)
    x = x_ref[...]
    xb = x_ref[pl.ds(i * BI, BI), :]
    piT = jax.lax.dot_general(wsi_ref[...], xb, (((0,), (1,)), ((), ())),
                              preferred_element_type=F32, precision=jax.lax.Precision.HIGHEST) + bs_ref[...]
    pjT = jax.lax.dot_general(wsj_ref[...], x, (((0,), (1,)), ((), ())),
                              preferred_element_type=F32, precision=jax.lax.Precision.HIGHEST)

    e_blk = e_row_ref[...]
    et_blk = e_col_ref[...].T
    T, wi, wo = _edge_block(piT, pjT, e_blk, et_blk,
                            u_ref[...], v_ref[...],
                            wai_ref[...], bai_ref[...],
                            wao_ref[...], bao_ref[...], a_ref[...])

    min_ref[...] = jnp.sum(T * wi[None, :, :], axis=2)
    mo = jnp.sum(T * wo[None, :, :], axis=1)

    @pl.when(i == 0)
    def _():
        mout_ref[...] = mo

    @pl.when(i > 0)
    def _():
        mout_ref[...] = mout_ref[...] + mo

    e_new = jnp.sum(T * we_ref[...], axis=0) + be_ref[...]
    e1_ref[...] = e_new
    e1t_ref[...] = e_new.T


def _layer2_kernel(x_ref, min1_ref, mout1_ref, wn1_ref, bn1_ref,
                   e1_ref, e1t_ref, a_ref,
                   wsi_ref, wsj_ref, u_ref, v_ref, bs_ref,
                   wai_ref, bai_ref, wao_ref, bao_ref,
                   wn2_ref, bn2_ref, wd_ref, bd_ref,
                   out_ref,
                   h1_scr, piT_scr, pjT_scr, min2_scr, mout2_scr):
    i = pl.program_id(0)
    nsteps = pl.num_programs(0)

    @pl.when(i == 0)
    def _():
        wn1 = wn1_ref[...]
        h1 = (
            jnp.dot(x_ref[...], wn1[:64], preferred_element_type=F32, precision=jax.lax.Precision.HIGHEST)
            + jax.lax.dot_general(min1_ref[...], wn1[64:96],
                                  (((0,), (0,)), ((), ())),
                                  preferred_element_type=F32, precision=jax.lax.Precision.HIGHEST)
            + jax.lax.dot_general(mout1_ref[...], wn1[96:128],
                                  (((0,), (0,)), ((), ())),
                                  preferred_element_type=F32, precision=jax.lax.Precision.HIGHEST)
            + bn1_ref[...]
        )
        h1_scr[...] = h1
        piT_scr[...] = jax.lax.dot_general(wsi_ref[...], h1,
                                           (((0,), (1,)), ((), ())),
                                           preferred_element_type=F32, precision=jax.lax.Precision.HIGHEST) + bs_ref[...]
        pjT_scr[...] = jax.lax.dot_general(wsj_ref[...], h1,
                                           (((0,), (1,)), ((), ())),
                                           preferred_element_type=F32, precision=jax.lax.Precision.HIGHEST)

    piT_blk = piT_scr[:, pl.ds(i * BI, BI)]
    T, wi, wo = _edge_block(piT_blk, pjT_scr[...],
                            e1_ref[...], e1t_ref[...],
                            u_ref[...], v_ref[...],
                            wai_ref[...], bai_ref[...],
                            wao_ref[...], bao_ref[...], a_ref[...])

    min2_scr[:, pl.ds(i * BI, BI)] = jnp.sum(T * wi[None, :, :], axis=2)
    mo = jnp.sum(T * wo[None, :, :], axis=1)

    @pl.when(i == 0)
    def _():
        mout2_scr[...] = mo

    @pl.when(i > 0)
    def _():
        mout2_scr[...] = mout2_scr[...] + mo

    @pl.when(i == nsteps - 1)
    def _():
        wn2 = wn2_ref[...]
        x2 = (
            jnp.dot(h1_scr[...], wn2[:240], preferred_element_type=F32, precision=jax.lax.Precision.HIGHEST)
            + jax.lax.dot_general(min2_scr[...], wn2[240:272],
                                  (((0,), (0,)), ((), ())),
                                  preferred_element_type=F32, precision=jax.lax.Precision.HIGHEST)
            + jax.lax.dot_general(mout2_scr[...], wn2[272:304],
                                  (((0,), (0,)), ((), ())),
                                  preferred_element_type=F32, precision=jax.lax.Precision.HIGHEST)
            + bn2_ref[...]
        )
        out_ref[...] = jnp.dot(x2, wd_ref[...],
                               preferred_element_type=F32, precision=jax.lax.Precision.HIGHEST) + bd_ref[...]


def _full(shape):
    return pl.BlockSpec(shape, lambda i: tuple(0 for _ in shape))


def kernel(x, a, e, Ws1, bs1, Wai1, bai1, Wao1, bao1, Wn1, bn1, We1, be1,
           Ws2, bs2, Wai2, bai2, Wao2, bao2, Wn2, bn2, We2, be2, Wd, bd):
    f = x.shape[-1]
    x2d = x.reshape(N, f)
    a2d = a.reshape(N, N)
    e2d = e.reshape(N, N)
    grid = (N // BI,)

    def prep(Ws, bs, Wai, bai, Wao, bao, d):
        return (Ws[:d], Ws[d:2 * d],
                Ws[2 * d].reshape(32, 1, 1), Ws[2 * d + 1].reshape(32, 1, 1),
                bs.reshape(32, 1),
                Wai.reshape(32, 1, 1), bai.reshape(1, 1),
                Wao.reshape(32, 1, 1), bao.reshape(1, 1))

    w1 = prep(Ws1, bs1, Wai1, bai1, Wao1, bao1, f)
    w2 = prep(Ws2, bs2, Wai2, bai2, Wao2, bao2, 240)

    row_spec = pl.BlockSpec((BI, N), lambda i: (i, 0))
    col_spec = pl.BlockSpec((N, BI), lambda i: (0, i))
    cblk_spec = pl.BlockSpec((32, BI), lambda i: (0, i))

    min1, mout1, e1, e1t = pl.pallas_call(
        _layer1_kernel,
        grid=grid,
        in_specs=[
            _full((N, f)), row_spec, col_spec, row_spec,
            _full((f, 32)), _full((f, 32)),
            _full((32, 1, 1)), _full((32, 1, 1)), _full((32, 1)),
            _full((32, 1, 1)), _full((1, 1)),
            _full((32, 1, 1)), _full((1, 1)),
            _full((32, 1, 1)), _full((1, 1)),
        ],
        out_specs=[cblk_spec, _full((32, N)), row_spec, col_spec],
        out_shape=[
            jax.ShapeDtypeStruct((32, N), F32),
            jax.ShapeDtypeStruct((32, N), F32),
            jax.ShapeDtypeStruct((N, N), F32),
            jax.ShapeDtypeStruct((N, N), F32),
        ],
    )(x2d, e2d, e2d, a2d,
      w1[0], w1[1], w1[2], w1[3], w1[4], w1[5], w1[6], w1[7], w1[8],
      We1.reshape(32, 1, 1), be1.reshape(1, 1))

    out = pl.pallas_call(
        _layer2_kernel,
        grid=grid,
        in_specs=[
            _full((N, f)), _full((32, N)), _full((32, N)),
            _full((f + 64, 240)), _full((1, 240)),
            row_spec, row_spec, row_spec,
            _full((240, 32)), _full((240, 32)),
            _full((32, 1, 1)), _full((32, 1, 1)), _full((32, 1)),
            _full((32, 1, 1)), _full((1, 1)),
            _full((32, 1, 1)), _full((1, 1)),
            _full((304, 240)), _full((1, 240)),
            _full((240, 240)), _full((1, 240)),
        ],
        out_specs=_full((N, 240)),
        out_shape=jax.ShapeDtypeStruct((N, 240), F32),
        scratch_shapes=[
            pltpu.VMEM((N, 240), F32),
            pltpu.VMEM((32, N), F32),
            pltpu.VMEM((32, N), F32),
            pltpu.VMEM((32, N), F32),
            pltpu.VMEM((32, N), F32),
        ],
    )(x2d, min1, mout1, Wn1, bn1.reshape(1, 240),
      e1, e1t, a2d,
      w2[0], w2[1], w2[2], w2[3], w2[4], w2[5], w2[6], w2[7], w2[8],
      Wn2, bn2.reshape(1, 240), Wd, bd.reshape(1, 240))

    return out.reshape(1, N, 240)


# speedup vs baseline: 1.9497x; 1.0716x over previous
import jax
import jax.numpy as jnp
from jax.experimental import pallas as pl
from jax.experimental.pallas import tpu as pltpu

N = 512
BI = 128
NB = N // BI
F32 = jnp.float32


def _edge_block(piT_blk, pjT, e_blk, et_blk, u, v, wai, bai, wao, bao, a_blk):
    T = jax.nn.relu(
        piT_blk[:, :, None]
        + pjT[:, None, :]
        + e_blk[None, :, :] * u
        + et_blk[None, :, :] * v
    )
    zi = jnp.sum(T * wai, axis=0) + bai
    zo = jnp.sum(T * wao, axis=0) + bao
    mask = (a_blk != 0.0).astype(F32)
    wi = mask * jax.nn.sigmoid(zi)
    wo = mask * jax.nn.sigmoid(zo)
    return T, wi, wo


def _fused_kernel(x_ref, e_row_ref, e_col_ref, a_ref,
                  wsi1_ref, wsj1_ref, u1_ref, v1_ref, bs1_ref,
                  wai1_ref, bai1_ref, wao1_ref, bao1_ref, we1_ref, be1_ref,
                  wn1_ref, bn1_ref,
                  wsi2_ref, wsj2_ref, u2_ref, v2_ref, bs2_ref,
                  wai2_ref, bai2_ref, wao2_ref, bao2_ref,
                  wn2_ref, bn2_ref, wd_ref, bd_ref,
                  out_ref,
                  min1_scr, mout1_scr, e1_scr, e1t_scr,
                  h1_scr, piT_scr, pjT_scr, min2_scr, mout2_scr):
    s = pl.program_id(0)
    i = s % NB
    ib = i * BI

    @pl.when(s < NB)
    def _layer1():
        piT = jax.lax.dot_general(
            wsi1_ref[...], x_ref[pl.ds(ib, BI), :], (((0,), (1,)), ((), ())),
            preferred_element_type=F32) + bs1_ref[...]
        pjT = jax.lax.dot_general(
            wsj1_ref[...], x_ref[...], (((0,), (1,)), ((), ())),
            preferred_element_type=F32)
        T, wi, wo = _edge_block(piT, pjT,
                                e_row_ref[...], e_col_ref[...].T,
                                u1_ref[...], v1_ref[...],
                                wai1_ref[...], bai1_ref[...],
                                wao1_ref[...], bao1_ref[...], a_ref[...])
        min1_scr[:, pl.ds(ib, BI)] = jnp.sum(T * wi[None, :, :], axis=2)
        mo = jnp.sum(T * wo[None, :, :], axis=1)

        @pl.when(s == 0)
        def _():
            mout1_scr[...] = mo

        @pl.when(s > 0)
        def _():
            mout1_scr[...] = mout1_scr[...] + mo

        e_new = jnp.sum(T * we1_ref[...], axis=0) + be1_ref[...]
        e1_scr[pl.ds(ib, BI), :] = e_new
        e1t_scr[:, pl.ds(ib, BI)] = e_new.T

    @pl.when(s == NB)
    def _node_update():
        wn1 = wn1_ref[...]
        h1 = (
            jnp.dot(x_ref[...], wn1[:64], preferred_element_type=F32)
            + jax.lax.dot_general(min1_scr[...], wn1[64:96],
                                  (((0,), (0,)), ((), ())),
                                  preferred_element_type=F32)
            + jax.lax.dot_general(mout1_scr[...], wn1[96:128],
                                  (((0,), (0,)), ((), ())),
                                  preferred_element_type=F32)
            + bn1_ref[...]
        )
        h1_scr[...] = h1
        piT_scr[...] = jax.lax.dot_general(wsi2_ref[...], h1,
                                           (((0,), (1,)), ((), ())),
                                           preferred_element_type=F32) + bs2_ref[...]
        pjT_scr[...] = jax.lax.dot_general(wsj2_ref[...], h1,
                                           (((0,), (1,)), ((), ())),
                                           preferred_element_type=F32)

    @pl.when(s >= NB)
    def _layer2():
        T, wi, wo = _edge_block(piT_scr[:, pl.ds(ib, BI)], pjT_scr[...],
                                e1_scr[pl.ds(ib, BI), :],
                                e1t_scr[pl.ds(ib, BI), :],
                                u2_ref[...], v2_ref[...],
                                wai2_ref[...], bai2_ref[...],
                                wao2_ref[...], bao2_ref[...], a_ref[...])
        min2_scr[:, pl.ds(ib, BI)] = jnp.sum(T * wi[None, :, :], axis=2)
        mo = jnp.sum(T * wo[None, :, :], axis=1)

        @pl.when(s == NB)
        def _():
            mout2_scr[...] = mo

        @pl.when(s > NB)
        def _():
            mout2_scr[...] = mout2_scr[...] + mo

    @pl.when(s == 2 * NB - 1)
    def _head():
        wn2 = wn2_ref[...]
        x2 = (
            jnp.dot(h1_scr[...], wn2[:240], preferred_element_type=F32)
            + jax.lax.dot_general(min2_scr[...], wn2[240:272],
                                  (((0,), (0,)), ((), ())),
                                  preferred_element_type=F32)
            + jax.lax.dot_general(mout2_scr[...], wn2[272:304],
                                  (((0,), (0,)), ((), ())),
                                  preferred_element_type=F32)
            + bn2_ref[...]
        )
        out_ref[...] = jnp.dot(x2, wd_ref[...],
                               preferred_element_type=F32) + bd_ref[...]


def _full(shape):
    return pl.BlockSpec(shape, lambda s: tuple(0 for _ in shape))


def kernel(x, a, e, Ws1, bs1, Wai1, bai1, Wao1, bao1, Wn1, bn1, We1, be1,
           Ws2, bs2, Wai2, bai2, Wao2, bao2, Wn2, bn2, We2, be2, Wd, bd):
    f = x.shape[-1]
    x2d = x.reshape(N, f)
    a2d = a.reshape(N, N)
    e2d = e.reshape(N, N)

    def prep(Ws, bs, Wai, bai, Wao, bao, d):
        return (Ws[:d], Ws[d:2 * d],
                Ws[2 * d].reshape(32, 1, 1), Ws[2 * d + 1].reshape(32, 1, 1),
                bs.reshape(32, 1),
                Wai.reshape(32, 1, 1), bai.reshape(1, 1),
                Wao.reshape(32, 1, 1), bao.reshape(1, 1))

    w1 = prep(Ws1, bs1, Wai1, bai1, Wao1, bao1, f)
    w2 = prep(Ws2, bs2, Wai2, bai2, Wao2, bao2, 240)

    row_spec = pl.BlockSpec((BI, N), lambda s: (s % NB, 0))
    col_spec = pl.BlockSpec((N, BI), lambda s: (0, s % NB))

    out = pl.pallas_call(
        _fused_kernel,
        grid=(2 * NB,),
        in_specs=[
            _full((N, f)), row_spec, col_spec, row_spec,
            _full((f, 32)), _full((f, 32)),
            _full((32, 1, 1)), _full((32, 1, 1)), _full((32, 1)),
            _full((32, 1, 1)), _full((1, 1)),
            _full((32, 1, 1)), _full((1, 1)),
            _full((32, 1, 1)), _full((1, 1)),
            _full((f + 64, 240)), _full((1, 240)),
            _full((240, 32)), _full((240, 32)),
            _full((32, 1, 1)), _full((32, 1, 1)), _full((32, 1)),
            _full((32, 1, 1)), _full((1, 1)),
            _full((32, 1, 1)), _full((1, 1)),
            _full((304, 240)), _full((1, 240)),
            _full((240, 240)), _full((1, 240)),
        ],
        out_specs=_full((N, 240)),
        out_shape=jax.ShapeDtypeStruct((N, 240), F32),
        scratch_shapes=[
            pltpu.VMEM((32, N), F32),
            pltpu.VMEM((32, N), F32),
            pltpu.VMEM((N, N), F32),
            pltpu.VMEM((N, N), F32),
            pltpu.VMEM((N, 240), F32),
            pltpu.VMEM((32, N), F32),
            pltpu.VMEM((32, N), F32),
            pltpu.VMEM((32, N), F32),
            pltpu.VMEM((32, N), F32),
        ],
    )(x2d, e2d, e2d, a2d,
      w1[0], w1[1], w1[2], w1[3], w1[4], w1[5], w1[6], w1[7], w1[8],
      We1.reshape(32, 1, 1), be1.reshape(1, 1),
      Wn1, bn1.reshape(1, 240),
      w2[0], w2[1], w2[2], w2[3], w2[4], w2[5], w2[6], w2[7], w2[8],
      Wn2, bn2.reshape(1, 240), Wd, bd.reshape(1, 240))

    return out.reshape(1, N, 240)


# speedup vs baseline: 2.0028x; 1.0272x over previous
import jax
import jax.numpy as jnp
from jax.experimental import pallas as pl
from jax.experimental.pallas import tpu as pltpu

N = 512
BI = 256
NB = N // BI
F32 = jnp.float32


def _edge_block(piT_blk, pjT, e_blk, et_blk, u, v, wai, bai, wao, bao, a_blk):
    T = jax.nn.relu(
        piT_blk[:, :, None]
        + pjT[:, None, :]
        + e_blk[None, :, :] * u
        + et_blk[None, :, :] * v
    )
    zi = jnp.sum(T * wai, axis=0) + bai
    zo = jnp.sum(T * wao, axis=0) + bao
    mask = (a_blk != 0.0).astype(F32)
    wi = mask * jax.nn.sigmoid(zi)
    wo = mask * jax.nn.sigmoid(zo)
    return T, wi, wo


def _fused_kernel(x_ref, e_row_ref, e_col_ref, a_ref,
                  wsi1_ref, wsj1_ref, u1_ref, v1_ref, bs1_ref,
                  wai1_ref, bai1_ref, wao1_ref, bao1_ref, we1_ref, be1_ref,
                  wn1_ref, bn1_ref,
                  wsi2_ref, wsj2_ref, u2_ref, v2_ref, bs2_ref,
                  wai2_ref, bai2_ref, wao2_ref, bao2_ref,
                  wn2_ref, bn2_ref, wd_ref, bd_ref,
                  out_ref,
                  min1_scr, mout1_scr, e1_scr, e1t_scr,
                  h1_scr, piT_scr, pjT_scr, min2_scr, mout2_scr):
    s = pl.program_id(0)
    i = s % NB
    ib = i * BI

    @pl.when(s < NB)
    def _layer1():
        piT = jax.lax.dot_general(
            wsi1_ref[...], x_ref[pl.ds(ib, BI), :], (((0,), (1,)), ((), ())),
            preferred_element_type=F32) + bs1_ref[...]
        pjT = jax.lax.dot_general(
            wsj1_ref[...], x_ref[...], (((0,), (1,)), ((), ())),
            preferred_element_type=F32)
        T, wi, wo = _edge_block(piT, pjT,
                                e_row_ref[...], e_col_ref[...].T,
                                u1_ref[...], v1_ref[...],
                                wai1_ref[...], bai1_ref[...],
                                wao1_ref[...], bao1_ref[...], a_ref[...])
        min1_scr[:, pl.ds(ib, BI)] = jnp.sum(T * wi[None, :, :], axis=2)
        mo = jnp.sum(T * wo[None, :, :], axis=1)

        @pl.when(s == 0)
        def _():
            mout1_scr[...] = mo

        @pl.when(s > 0)
        def _():
            mout1_scr[...] = mout1_scr[...] + mo

        e_new = jnp.sum(T * we1_ref[...], axis=0) + be1_ref[...]
        e1_scr[pl.ds(ib, BI), :] = e_new
        e1t_scr[:, pl.ds(ib, BI)] = e_new.T

    @pl.when(s == NB)
    def _node_update():
        wn1 = wn1_ref[...]
        h1 = (
            jnp.dot(x_ref[...], wn1[:64], preferred_element_type=F32)
            + jax.lax.dot_general(min1_scr[...], wn1[64:96],
                                  (((0,), (0,)), ((), ())),
                                  preferred_element_type=F32)
            + jax.lax.dot_general(mout1_scr[...], wn1[96:128],
                                  (((0,), (0,)), ((), ())),
                                  preferred_element_type=F32)
            + bn1_ref[...]
        )
        h1_scr[...] = h1
        piT_scr[...] = jax.lax.dot_general(wsi2_ref[...], h1,
                                           (((0,), (1,)), ((), ())),
                                           preferred_element_type=F32) + bs2_ref[...]
        pjT_scr[...] = jax.lax.dot_general(wsj2_ref[...], h1,
                                           (((0,), (1,)), ((), ())),
                                           preferred_element_type=F32)

    @pl.when(s >= NB)
    def _layer2():
        T, wi, wo = _edge_block(piT_scr[:, pl.ds(ib, BI)], pjT_scr[...],
                                e1_scr[pl.ds(ib, BI), :],
                                e1t_scr[pl.ds(ib, BI), :],
                                u2_ref[...], v2_ref[...],
                                wai2_ref[...], bai2_ref[...],
                                wao2_ref[...], bao2_ref[...], a_ref[...])
        min2_scr[:, pl.ds(ib, BI)] = jnp.sum(T * wi[None, :, :], axis=2)
        mo = jnp.sum(T * wo[None, :, :], axis=1)

        @pl.when(s == NB)
        def _():
            mout2_scr[...] = mo

        @pl.when(s > NB)
        def _():
            mout2_scr[...] = mout2_scr[...] + mo

    @pl.when(s == 2 * NB - 1)
    def _head():
        wn2 = wn2_ref[...]
        x2 = (
            jnp.dot(h1_scr[...], wn2[:240], preferred_element_type=F32)
            + jax.lax.dot_general(min2_scr[...], wn2[240:272],
                                  (((0,), (0,)), ((), ())),
                                  preferred_element_type=F32)
            + jax.lax.dot_general(mout2_scr[...], wn2[272:304],
                                  (((0,), (0,)), ((), ())),
                                  preferred_element_type=F32)
            + bn2_ref[...]
        )
        out_ref[...] = jnp.dot(x2, wd_ref[...],
                               preferred_element_type=F32) + bd_ref[...]


def _full(shape):
    return pl.BlockSpec(shape, lambda s: tuple(0 for _ in shape))


def kernel(x, a, e, Ws1, bs1, Wai1, bai1, Wao1, bao1, Wn1, bn1, We1, be1,
           Ws2, bs2, Wai2, bai2, Wao2, bao2, Wn2, bn2, We2, be2, Wd, bd):
    f = x.shape[-1]
    x2d = x.reshape(N, f)
    a2d = a.reshape(N, N)
    e2d = e.reshape(N, N)

    def prep(Ws, bs, Wai, bai, Wao, bao, d):
        return (Ws[:d], Ws[d:2 * d],
                Ws[2 * d].reshape(32, 1, 1), Ws[2 * d + 1].reshape(32, 1, 1),
                bs.reshape(32, 1),
                Wai.reshape(32, 1, 1), bai.reshape(1, 1),
                Wao.reshape(32, 1, 1), bao.reshape(1, 1))

    w1 = prep(Ws1, bs1, Wai1, bai1, Wao1, bao1, f)
    w2 = prep(Ws2, bs2, Wai2, bai2, Wao2, bao2, 240)

    row_spec = pl.BlockSpec((BI, N), lambda s: (s % NB, 0))
    col_spec = pl.BlockSpec((N, BI), lambda s: (0, s % NB))

    out = pl.pallas_call(
        _fused_kernel,
        grid=(2 * NB,),
        in_specs=[
            _full((N, f)), row_spec, col_spec, row_spec,
            _full((f, 32)), _full((f, 32)),
            _full((32, 1, 1)), _full((32, 1, 1)), _full((32, 1)),
            _full((32, 1, 1)), _full((1, 1)),
            _full((32, 1, 1)), _full((1, 1)),
            _full((32, 1, 1)), _full((1, 1)),
            _full((f + 64, 240)), _full((1, 240)),
            _full((240, 32)), _full((240, 32)),
            _full((32, 1, 1)), _full((32, 1, 1)), _full((32, 1)),
            _full((32, 1, 1)), _full((1, 1)),
            _full((32, 1, 1)), _full((1, 1)),
            _full((304, 240)), _full((1, 240)),
            _full((240, 240)), _full((1, 240)),
        ],
        out_specs=_full((N, 240)),
        out_shape=jax.ShapeDtypeStruct((N, 240), F32),
        scratch_shapes=[
            pltpu.VMEM((32, N), F32),
            pltpu.VMEM((32, N), F32),
            pltpu.VMEM((N, N), F32),
            pltpu.VMEM((N, N), F32),
            pltpu.VMEM((N, 240), F32),
            pltpu.VMEM((32, N), F32),
            pltpu.VMEM((32, N), F32),
            pltpu.VMEM((32, N), F32),
            pltpu.VMEM((32, N), F32),
        ],
    )(x2d, e2d, e2d, a2d,
      w1[0], w1[1], w1[2], w1[3], w1[4], w1[5], w1[6], w1[7], w1[8],
      We1.reshape(32, 1, 1), be1.reshape(1, 1),
      Wn1, bn1.reshape(1, 240),
      w2[0], w2[1], w2[2], w2[3], w2[4], w2[5], w2[6], w2[7], w2[8],
      Wn2, bn2.reshape(1, 240), Wd, bd.reshape(1, 240))

    return out.reshape(1, N, 240)
